# baseline (speedup 1.0000x reference)
import jax
import jax.numpy as jnp
from jax.experimental import pallas as pl

_NODE = 32
_KNN = 5
_B = 2
_DN = (((1,), (1,)), ((), ()))
_DT = (((0,), (0,)), ((), ()))


def _stage_a(x_ref, eg_ref, w0_ref, g0_ref, b0_ref, anc_ref, sp_ref,
             fi_ref, xt_ref, feat_ref):
    _, C, N = x_ref.shape
    BN = _B * N
    K2 = _B * _NODE
    x1s = []
    for b in range(_B):
        xtb = x_ref[b]
        x1t = jax.nn.sigmoid(eg_ref[b]) * xtb
        xt_ref[b * N:(b + 1) * N, :] = jnp.transpose(xtb)
        x1s.append(jnp.transpose(x1t))
    x1 = jnp.concatenate(x1s, axis=0)
    h = jax.lax.dot_general(x1, w0_ref[...], _DN,
                            preferred_element_type=jnp.float32)
    cnt = float(BN)
    s1 = h.sum(axis=0, keepdims=True) / cnt
    s2 = (h * h).sum(axis=0, keepdims=True) / cnt
    var = s2 - s1 * s1
    inv = jax.lax.rsqrt(var + 1e-5)
    g0 = g0_ref[...] * inv
    b0 = b0_ref[...] - s1 * g0
    z = jnp.maximum(h * g0 + b0, 0.0)

    sig = jax.nn.sigmoid(sp_ref[...])
    anc = anc_ref[...]
    inv_s2h = -0.5 / (sig * sig)
    a_is2 = anc * inv_s2h * -2.0
    ones_c = jnp.ones((1, C), jnp.float32)
    const = jax.lax.dot_general(ones_c, anc * anc * inv_s2h, _DN,
                                preferred_element_type=jnp.float32)
    q = jax.lax.dot_general(z * z, inv_s2h, _DN,
                            preferred_element_type=jnp.float32)
    lx = jax.lax.dot_general(z, a_is2, _DN,
                             preferred_element_type=jnp.float32)
    logits = q + lx + const
    m = logits.max(axis=1, keepdims=True)
    e = jnp.exp(logits - m)
    sa = e / e.sum(axis=1, keepdims=True)
    ones_n = jnp.ones((N, 1), jnp.float32)
    feats = []
    for b in range(_B):
        sab = sa[b * N:(b + 1) * N]
        zb = z[b * N:(b + 1) * N]
        den = jax.lax.dot_general(sab, ones_n, _DT,
                                  preferred_element_type=jnp.float32)
        t = jax.lax.dot_general(sab, zb, _DT,
                                preferred_element_type=jnp.float32)
        nodes = (t - anc * den) / sig / (den + 1e-9)
        rn = jnp.sqrt((nodes * nodes).sum(axis=1, keepdims=True))
        nodes = nodes / jnp.maximum(rn, 1e-12)
        gn = jnp.sqrt((nodes * nodes).sum(keepdims=True))
        nodes = nodes / jnp.maximum(gn, 1e-12)
        cols = [jnp.transpose(nodes[:, _NODE * u:_NODE * (u + 1)])[:, :, None]
                for u in range(C // _NODE)]
        feat = jnp.concatenate(cols, axis=2).reshape(_NODE, C)
        feat_ref[b * _NODE:(b + 1) * _NODE, :] = feat
        feats.append(feat)
    f2 = jnp.concatenate(feats, axis=0)

    xt = jnp.concatenate(
        [jnp.transpose(x_ref[b]) for b in range(_B)], axis=0)
    xs = (xt * xt).sum(axis=1, keepdims=True)
    fs = jax.lax.dot_general(ones_c, f2 * f2, _DN,
                             preferred_element_type=jnp.float32)
    xdf = jax.lax.dot_general(xt, f2, _DN,
                              preferred_element_type=jnp.float32)
    d2 = xs - 2.0 * xdf + fs
    lane = jax.lax.broadcasted_iota(jnp.int32, (BN, K2), 1)
    row = jax.lax.broadcasted_iota(jnp.int32, (BN, K2), 0)
    other = (lane // _NODE) != (row // N)
    inf = jnp.float32(jnp.inf)
    d2 = jnp.where(other, inf, d2)
    for r in range(_KNN):
        mn = d2.min(axis=1, keepdims=True)
        cand = jnp.where(d2 == mn, lane, K2)
        fi = cand.min(axis=1, keepdims=True)
        d2 = jnp.where(lane == fi, inf, d2)
        for b in range(_B):
            fi_ref[b, r] = fi[b * N:(b + 1) * N]


def _stage_b2(xt_ref, f_ref, w1_ref, fi5_ref, g1_ref, b1_ref, out_ref):
    BN, C = xt_ref.shape
    K2 = _B * _NODE
    w1 = w1_ref[...]
    w1a = w1[:, :C]
    wd = w1[:, C:] - w1a
    xt = xt_ref[...]
    f2 = f_ref[...]
    g2 = jax.lax.dot_general(f2, w1a, _DN,
                             preferred_element_type=jnp.float32)
    p = jax.lax.dot_general(xt, wd, _DN,
                            preferred_element_type=jnp.float32)
    lane = jax.lax.broadcasted_iota(jnp.int32, (BN, K2), 1)
    fi5 = fi5_ref[...]
    gmax = None
    cnt = None
    for j in range(_KNN):
        col = fi5[:, j:j + 1]
        ohc = (lane == col).astype(jnp.float32)
        cnt = ohc if cnt is None else cnt + ohc
        gsel = jnp.dot(ohc, g2, preferred_element_type=jnp.float32)
        gmax = gsel if gmax is None else jnp.maximum(gmax, gsel)
    g2sq = g2 * g2
    cnt_tot = cnt.sum(axis=0, keepdims=True)
    q = jax.lax.dot_general(cnt, p, _DT,
                            preferred_element_type=jnp.float32)
    s1 = _KNN * p.sum(axis=0, keepdims=True) + jnp.dot(
        cnt_tot, g2, preferred_element_type=jnp.float32)
    s2 = (_KNN * (p * p).sum(axis=0, keepdims=True)
          + jnp.dot(cnt_tot, g2sq, preferred_element_type=jnp.float32)
          + 2.0 * (g2 * q).sum(axis=0, keepdims=True))
    tot = float(BN * _KNN)
    mean = s1 / tot
    var = s2 / tot - mean * mean
    a1 = g1_ref[...] * jax.lax.rsqrt(var + 1e-5)
    b1 = b1_ref[...] - a1 * mean
    y = jnp.maximum(a1 * (gmax + p) + b1, 0.0)
    o = xt + y
    N = BN // _B
    for b in range(_B):
        out_ref[b] = jnp.transpose(o[b * N:(b + 1) * N])


@jax.jit
def kernel(x, edge, W0, gamma0, beta0, anchor, sigma_p, W1, gamma1, beta1):
    B, C, H, W = x.shape
    N = H * W
    x3 = x.reshape(B, C, N)
    eg3 = edge.reshape(B, 1, N)
    fi_stack, xt, nodes_feat = pl.pallas_call(
        _stage_a,
        out_shape=(jax.ShapeDtypeStruct((B, _KNN, N, 1), jnp.int32),
                   jax.ShapeDtypeStruct((B * N, C), jnp.float32),
                   jax.ShapeDtypeStruct((B * _NODE, C), jnp.float32)),
    )(x3, eg3, W0, gamma0[None], beta0[None], anchor, sigma_p)
    fi5 = fi_stack.reshape(B, _KNN * N).reshape(B, N, _KNN).reshape(
        B * N, _KNN)
    out3 = pl.pallas_call(
        _stage_b2,
        out_shape=jax.ShapeDtypeStruct((B, C, N), jnp.float32),
    )(xt, nodes_feat, W1, fi5, gamma1[None], beta1[None])
    return out3.reshape(B, C, H, W)


# speedup vs baseline: 1.2157x; 1.2157x over previous
import jax
import jax.numpy as jnp
from jax.experimental import pallas as pl

_NODE = 32
_KNN = 5
_B = 2
_DN = (((1,), (1,)), ((), ()))
_DT = (((0,), (0,)), ((), ()))


def _stage_a(x_ref, eg_ref, w0_ref, g0_ref, b0_ref, anc_ref, sp_ref,
             fi_ref, xt_ref, feat_ref):
    _, C, N = x_ref.shape
    BN = _B * N
    K2 = _B * _NODE
    x1s = []
    for b in range(_B):
        xtb = x_ref[b]
        x1t = jax.nn.sigmoid(eg_ref[b]) * xtb
        xt_ref[b * N:(b + 1) * N, :] = jnp.transpose(xtb)
        x1s.append(jnp.transpose(x1t))
    x1 = jnp.concatenate(x1s, axis=0)
    h = jax.lax.dot_general(x1, w0_ref[...], _DN,
                            preferred_element_type=jnp.float32)
    cnt = float(BN)
    s1 = h.sum(axis=0, keepdims=True) / cnt
    s2 = (h * h).sum(axis=0, keepdims=True) / cnt
    var = s2 - s1 * s1
    inv = jax.lax.rsqrt(var + 1e-5)
    g0 = g0_ref[...] * inv
    b0 = b0_ref[...] - s1 * g0
    z = jnp.maximum(h * g0 + b0, 0.0)

    sig = jax.nn.sigmoid(sp_ref[...])
    anc = anc_ref[...]
    inv_s2h = -0.5 / (sig * sig)
    a_is2 = anc * inv_s2h * -2.0
    ones_c = jnp.ones((1, C), jnp.float32)
    const = jax.lax.dot_general(ones_c, anc * anc * inv_s2h, _DN,
                                preferred_element_type=jnp.float32)
    q = jax.lax.dot_general(z * z, inv_s2h, _DN,
                            preferred_element_type=jnp.float32)
    lx = jax.lax.dot_general(z, a_is2, _DN,
                             preferred_element_type=jnp.float32)
    logits = q + lx + const
    m = logits.max(axis=1, keepdims=True)
    e = jnp.exp(logits - m)
    sa = e / e.sum(axis=1, keepdims=True)
    ones_n = jnp.ones((N, 1), jnp.float32)
    feats = []
    for b in range(_B):
        sab = sa[b * N:(b + 1) * N]
        zb = z[b * N:(b + 1) * N]
        den = jax.lax.dot_general(sab, ones_n, _DT,
                                  preferred_element_type=jnp.float32)
        t = jax.lax.dot_general(sab, zb, _DT,
                                preferred_element_type=jnp.float32)
        nodes = (t - anc * den) / sig / (den + 1e-9)
        rn = jnp.sqrt((nodes * nodes).sum(axis=1, keepdims=True))
        nodes = nodes / jnp.maximum(rn, 1e-12)
        gn = jnp.sqrt((nodes * nodes).sum(keepdims=True))
        nodes = nodes / jnp.maximum(gn, 1e-12)
        cols = [jnp.transpose(nodes[:, _NODE * u:_NODE * (u + 1)])[:, :, None]
                for u in range(C // _NODE)]
        feat = jnp.concatenate(cols, axis=2).reshape(_NODE, C)
        feat_ref[b * _NODE:(b + 1) * _NODE, :] = feat
        feats.append(feat)
    f2 = jnp.concatenate(feats, axis=0)

    xt = jnp.concatenate(
        [jnp.transpose(x_ref[b]) for b in range(_B)], axis=0)
    xs = (xt * xt).sum(axis=1, keepdims=True)
    fs = jax.lax.dot_general(ones_c, f2 * f2, _DN,
                             preferred_element_type=jnp.float32)
    xdf = jax.lax.dot_general(xt, f2, _DN,
                              preferred_element_type=jnp.float32)
    d2 = xs - 2.0 * xdf + fs
    lane = jax.lax.broadcasted_iota(jnp.int32, (BN, K2), 1)
    row = jax.lax.broadcasted_iota(jnp.int32, (BN, K2), 0)
    other = (lane // _NODE) != (row // N)
    inf = jnp.float32(jnp.inf)
    d2 = jnp.where(other, inf, d2)
    fis = []
    for r in range(_KNN):
        mn = d2.min(axis=1, keepdims=True)
        cand = jnp.where(d2 == mn, lane, K2)
        fi = cand.min(axis=1, keepdims=True)
        d2 = jnp.where(lane == fi, inf, d2)
        fis.append(fi)
    fit = jnp.transpose(jnp.concatenate(fis, axis=1))
    for b in range(_B):
        fi_ref[b] = fit[:, b * N:(b + 1) * N]


def _stage_b2(xt_ref, f_ref, w1_ref, fi5_ref, g1_ref, b1_ref, out_ref):
    BN, C = xt_ref.shape
    K2 = _B * _NODE
    w1 = w1_ref[...]
    w1a = w1[:, :C]
    wd = w1[:, C:] - w1a
    xt = xt_ref[...]
    f2 = f_ref[...]
    g2 = jax.lax.dot_general(f2, w1a, _DN,
                             preferred_element_type=jnp.float32)
    p = jax.lax.dot_general(xt, wd, _DN,
                            preferred_element_type=jnp.float32)
    lane = jax.lax.broadcasted_iota(jnp.int32, (BN, K2), 1)
    fi5 = fi5_ref[...]
    gmax = None
    cnt = None
    for j in range(_KNN):
        col = fi5[:, j:j + 1]
        ohc = (lane == col).astype(jnp.float32)
        cnt = ohc if cnt is None else cnt + ohc
        gsel = jnp.dot(ohc, g2, preferred_element_type=jnp.float32)
        gmax = gsel if gmax is None else jnp.maximum(gmax, gsel)
    g2sq = g2 * g2
    cnt_tot = cnt.sum(axis=0, keepdims=True)
    q = jax.lax.dot_general(cnt, p, _DT,
                            preferred_element_type=jnp.float32)
    s1 = _KNN * p.sum(axis=0, keepdims=True) + jnp.dot(
        cnt_tot, g2, preferred_element_type=jnp.float32)
    s2 = (_KNN * (p * p).sum(axis=0, keepdims=True)
          + jnp.dot(cnt_tot, g2sq, preferred_element_type=jnp.float32)
          + 2.0 * (g2 * q).sum(axis=0, keepdims=True))
    tot = float(BN * _KNN)
    mean = s1 / tot
    var = s2 / tot - mean * mean
    a1 = g1_ref[...] * jax.lax.rsqrt(var + 1e-5)
    b1 = b1_ref[...] - a1 * mean
    y = jnp.maximum(a1 * (gmax + p) + b1, 0.0)
    o = xt + y
    N = BN // _B
    for b in range(_B):
        out_ref[b] = jnp.transpose(o[b * N:(b + 1) * N])


@jax.jit
def kernel(x, edge, W0, gamma0, beta0, anchor, sigma_p, W1, gamma1, beta1):
    B, C, H, W = x.shape
    N = H * W
    x3 = x.reshape(B, C, N)
    eg3 = edge.reshape(B, 1, N)
    fi_stack, xt, nodes_feat = pl.pallas_call(
        _stage_a,
        out_shape=(jax.ShapeDtypeStruct((B, _KNN, N), jnp.int32),
                   jax.ShapeDtypeStruct((B * N, C), jnp.float32),
                   jax.ShapeDtypeStruct((B * _NODE, C), jnp.float32)),
    )(x3, eg3, W0, gamma0[None], beta0[None], anchor, sigma_p)
    fi5 = fi_stack.reshape(B, _KNN * N).reshape(B, N, _KNN).reshape(
        B * N, _KNN)
    out3 = pl.pallas_call(
        _stage_b2,
        out_shape=jax.ShapeDtypeStruct((B, C, N), jnp.float32),
    )(xt, nodes_feat, W1, fi5, gamma1[None], beta1[None])
    return out3.reshape(B, C, H, W)


# speedup vs baseline: 1.4377x; 1.1826x over previous
import jax
import jax.numpy as jnp
from jax.experimental import pallas as pl

_NODE = 32
_KNN = 5
_B = 2
_DN = (((1,), (1,)), ((), ()))
_DT = (((0,), (0,)), ((), ()))


def _stage_a(x_ref, eg_ref, w0_ref, g0_ref, b0_ref, anc_ref, sp_ref,
             fi_ref, feat_ref):
    _, C, N = x_ref.shape
    BN = _B * N
    K2 = _B * _NODE
    x1s = []
    for b in range(_B):
        xtb = x_ref[b]
        x1t = jax.nn.sigmoid(eg_ref[b]) * xtb
        x1s.append(jnp.transpose(x1t))
    x1 = jnp.concatenate(x1s, axis=0)
    h = jax.lax.dot_general(x1, w0_ref[...], _DN,
                            preferred_element_type=jnp.float32)
    cnt = float(BN)
    s1 = h.sum(axis=0, keepdims=True) / cnt
    s2 = (h * h).sum(axis=0, keepdims=True) / cnt
    var = s2 - s1 * s1
    inv = jax.lax.rsqrt(var + 1e-5)
    g0 = g0_ref[...] * inv
    b0 = b0_ref[...] - s1 * g0
    z = jnp.maximum(h * g0 + b0, 0.0)

    sig = jax.nn.sigmoid(sp_ref[...])
    anc = anc_ref[...]
    inv_s2h = -0.5 / (sig * sig)
    a_is2 = anc * inv_s2h * -2.0
    ones_c = jnp.ones((1, C), jnp.float32)
    const = jax.lax.dot_general(ones_c, anc * anc * inv_s2h, _DN,
                                preferred_element_type=jnp.float32)
    q = jax.lax.dot_general(z * z, inv_s2h, _DN,
                            preferred_element_type=jnp.float32)
    lx = jax.lax.dot_general(z, a_is2, _DN,
                             preferred_element_type=jnp.float32)
    logits = q + lx + const
    m = logits.max(axis=1, keepdims=True)
    e = jnp.exp(logits - m)
    sa = e / e.sum(axis=1, keepdims=True)
    ones_n = jnp.ones((N, 1), jnp.float32)
    feats = []
    for b in range(_B):
        sab = sa[b * N:(b + 1) * N]
        zb = z[b * N:(b + 1) * N]
        den = jax.lax.dot_general(sab, ones_n, _DT,
                                  preferred_element_type=jnp.float32)
        t = jax.lax.dot_general(sab, zb, _DT,
                                preferred_element_type=jnp.float32)
        nodes = (t - anc * den) / sig / (den + 1e-9)
        rn = jnp.sqrt((nodes * nodes).sum(axis=1, keepdims=True))
        nodes = nodes / jnp.maximum(rn, 1e-12)
        gn = jnp.sqrt((nodes * nodes).sum(keepdims=True))
        nodes = nodes / jnp.maximum(gn, 1e-12)
        cols = [jnp.transpose(nodes[:, _NODE * u:_NODE * (u + 1)])[:, :, None]
                for u in range(C // _NODE)]
        feat = jnp.concatenate(cols, axis=2).reshape(_NODE, C)
        feat_ref[b * _NODE:(b + 1) * _NODE, :] = feat
        feats.append(feat)
    f2 = jnp.concatenate(feats, axis=0)

    xt = jnp.concatenate(
        [jnp.transpose(x_ref[b]) for b in range(_B)], axis=0)
    xs = (xt * xt).sum(axis=1, keepdims=True)
    fs = jax.lax.dot_general(ones_c, f2 * f2, _DN,
                             preferred_element_type=jnp.float32)
    xdf = jax.lax.dot_general(xt, f2, _DN,
                              preferred_element_type=jnp.float32)
    d2 = xs - 2.0 * xdf + fs
    lane = jax.lax.broadcasted_iota(jnp.int32, (BN, K2), 1)
    row = jax.lax.broadcasted_iota(jnp.int32, (BN, K2), 0)
    other = (lane // _NODE) != (row // N)
    inf = jnp.float32(jnp.inf)
    d2 = jnp.where(other, inf, d2)
    bits = jax.lax.bitcast_convert_type(jnp.maximum(d2, 0.0), jnp.int32)
    key = (bits & jnp.int32(~63)) | lane
    imax = jnp.int32(0x7FFFFFFF)
    fis = []
    for r in range(_KNN):
        mn = key.min(axis=1, keepdims=True)
        key = jnp.where(key == mn, imax, key)
        fis.append(mn & jnp.int32(63))
    fit = jnp.transpose(jnp.concatenate(fis, axis=1))
    for b in range(_B):
        fi_ref[b] = fit[:, b * N:(b + 1) * N]


def _stage_b2(x_ref, f_ref, w1_ref, fi5_ref, g1_ref, b1_ref, out_ref):
    _, C, N = x_ref.shape
    BN = _B * N
    K2 = _B * _NODE
    w1 = w1_ref[...]
    w1a = w1[:, :C]
    wd = w1[:, C:] - w1a
    xt = jnp.concatenate(
        [jnp.transpose(x_ref[b]) for b in range(_B)], axis=0)
    f2 = f_ref[...]
    g2 = jax.lax.dot_general(f2, w1a, _DN,
                             preferred_element_type=jnp.float32)
    p = jax.lax.dot_general(xt, wd, _DN,
                            preferred_element_type=jnp.float32)
    lane = jax.lax.broadcasted_iota(jnp.int32, (BN, K2), 1)
    fi5 = fi5_ref[...]
    gmax = None
    cnt = None
    for j in range(_KNN):
        col = fi5[:, j:j + 1]
        ohc = (lane == col).astype(jnp.float32)
        cnt = ohc if cnt is None else cnt + ohc
        gsel = jnp.dot(ohc, g2, preferred_element_type=jnp.float32)
        gmax = gsel if gmax is None else jnp.maximum(gmax, gsel)
    g2sq = g2 * g2
    cnt_tot = cnt.sum(axis=0, keepdims=True)
    q = jax.lax.dot_general(cnt, p, _DT,
                            preferred_element_type=jnp.float32)
    s1 = _KNN * p.sum(axis=0, keepdims=True) + jnp.dot(
        cnt_tot, g2, preferred_element_type=jnp.float32)
    s2 = (_KNN * (p * p).sum(axis=0, keepdims=True)
          + jnp.dot(cnt_tot, g2sq, preferred_element_type=jnp.float32)
          + 2.0 * (g2 * q).sum(axis=0, keepdims=True))
    tot = float(BN * _KNN)
    mean = s1 / tot
    var = s2 / tot - mean * mean
    a1 = g1_ref[...] * jax.lax.rsqrt(var + 1e-5)
    b1 = b1_ref[...] - a1 * mean
    y = jnp.maximum(a1 * (gmax + p) + b1, 0.0)
    o = xt + y
    for b in range(_B):
        out_ref[b] = jnp.transpose(o[b * N:(b + 1) * N])


@jax.jit
def kernel(x, edge, W0, gamma0, beta0, anchor, sigma_p, W1, gamma1, beta1):
    B, C, H, W = x.shape
    N = H * W
    x3 = x.reshape(B, C, N)
    eg3 = edge.reshape(B, 1, N)
    fi_stack, nodes_feat = pl.pallas_call(
        _stage_a,
        out_shape=(jax.ShapeDtypeStruct((B, _KNN, N), jnp.int32),
                   jax.ShapeDtypeStruct((B * _NODE, C), jnp.float32)),
    )(x3, eg3, W0, gamma0[None], beta0[None], anchor, sigma_p)
    fi5 = fi_stack.reshape(B, _KNN * N).reshape(B, N, _KNN).reshape(
        B * N, _KNN)
    out3 = pl.pallas_call(
        _stage_b2,
        out_shape=jax.ShapeDtypeStruct((B, C, N), jnp.float32),
    )(x3, nodes_feat, W1, fi5, gamma1[None], beta1[None])
    return out3.reshape(B, C, H, W)


# speedup vs baseline: 1.9558x; 1.3604x over previous
import jax
import jax.numpy as jnp
from jax.experimental import pallas as pl

_NODE = 32
_KNN = 5
_B = 2
_MM = (((1,), (0,)), ((), ()))
_DN = (((1,), (1,)), ((), ()))
_D0 = (((0,), (0,)), ((), ()))


def _stage_a(x_ref, eg_ref, w0_ref, g0_ref, b0_ref, anc_ref, sp_ref,
             fi_ref, feat_ref):
    _, C, N = x_ref.shape
    BN = _B * N
    K2 = _B * _NODE
    hs = []
    for b in range(_B):
        x1t = jax.nn.sigmoid(eg_ref[b]) * x_ref[b]
        hs.append(jax.lax.dot_general(w0_ref[...], x1t, _MM,
                                      preferred_element_type=jnp.float32))
    cnt = float(BN)
    s1 = sum(h.sum(axis=1, keepdims=True) for h in hs) / cnt
    s2 = sum((h * h).sum(axis=1, keepdims=True) for h in hs) / cnt
    var = s2 - s1 * s1
    inv = jax.lax.rsqrt(var + 1e-5)
    g0 = g0_ref[...] * inv
    b0 = b0_ref[...] - s1 * g0
    zs = [jnp.maximum(h * g0 + b0, 0.0) for h in hs]

    sig = jax.nn.sigmoid(sp_ref[...])
    anc = anc_ref[...]
    inv_s2h = -0.5 / (sig * sig)
    a_is2 = anc * inv_s2h * -2.0
    const = (anc * anc * inv_s2h).sum(axis=1, keepdims=True)
    feats = []
    for b in range(_B):
        zt = zs[b]
        q = jax.lax.dot_general(inv_s2h, zt * zt, _MM,
                                preferred_element_type=jnp.float32)
        lx = jax.lax.dot_general(a_is2, zt, _MM,
                                 preferred_element_type=jnp.float32)
        logits = q + lx + const
        m = logits.max(axis=0, keepdims=True)
        e = jnp.exp(logits - m)
        sa = e / e.sum(axis=0, keepdims=True)
        den = sa.sum(axis=1, keepdims=True)
        t = jax.lax.dot_general(sa, zt, _DN,
                                preferred_element_type=jnp.float32)
        nodes = (t - anc * den) / sig / (den + 1e-9)
        rn = jnp.sqrt((nodes * nodes).sum(axis=1, keepdims=True))
        nodes = nodes / jnp.maximum(rn, 1e-12)
        gn = jnp.sqrt((nodes * nodes).sum(keepdims=True))
        nodes = nodes / jnp.maximum(gn, 1e-12)
        cols = [jnp.transpose(nodes[:, _NODE * u:_NODE * (u + 1)])[:, :, None]
                for u in range(C // _NODE)]
        feat = jnp.concatenate(cols, axis=2).reshape(_NODE, C)
        feat_ref[b * _NODE:(b + 1) * _NODE, :] = feat
        feats.append(feat)
    f2 = jnp.concatenate(feats, axis=0)

    fsq = (f2 * f2).sum(axis=1, keepdims=True)
    d2s = []
    for b in range(_B):
        xb = x_ref[b]
        xsq = (xb * xb).sum(axis=0, keepdims=True)
        fx = jax.lax.dot_general(f2, xb, _MM,
                                 preferred_element_type=jnp.float32)
        d2s.append(fsq - 2.0 * fx + xsq)
    d2 = jnp.concatenate(d2s, axis=1)
    subl = jax.lax.broadcasted_iota(jnp.int32, (K2, BN), 0)
    lane = jax.lax.broadcasted_iota(jnp.int32, (K2, BN), 1)
    other = (subl // _NODE) != (lane // N)
    bits = jax.lax.bitcast_convert_type(jnp.maximum(d2, 0.0), jnp.int32)
    imax = jnp.int32(0x7FFFFFFF)
    key = jnp.where(other, imax, (bits & jnp.int32(~63)) | subl)
    fis = []
    for r in range(_KNN):
        mn = key.min(axis=0, keepdims=True)
        key = jnp.where(key == mn, imax, key)
        fis.append(mn & jnp.int32(63))
    fit = jnp.concatenate(fis, axis=0)
    for b in range(_B):
        fi_ref[b] = fit[:, b * N:(b + 1) * N]


def _stage_b2(x_ref, f_ref, w1_ref, fi5_ref, g1_ref, b1_ref, out_ref):
    _, C, N = x_ref.shape
    BN = _B * N
    K2 = _B * _NODE
    w1 = w1_ref[...]
    w1a = w1[:, :C]
    wd = w1[:, C:] - w1a
    f2 = f_ref[...]
    g2 = jax.lax.dot_general(f2, w1a, _DN,
                             preferred_element_type=jnp.float32)
    p = jnp.concatenate(
        [jax.lax.dot_general(wd, x_ref[b], _MM,
                             preferred_element_type=jnp.float32)
         for b in range(_B)], axis=1)
    subl = jax.lax.broadcasted_iota(jnp.int32, (K2, BN), 0)
    fi5 = fi5_ref[...]
    gmax = None
    cnt = None
    for j in range(_KNN):
        ohc = (subl == fi5[j:j + 1, :]).astype(jnp.float32)
        cnt = ohc if cnt is None else cnt + ohc
        gsel = jax.lax.dot_general(g2, ohc, _D0,
                                   preferred_element_type=jnp.float32)
        gmax = gsel if gmax is None else jnp.maximum(gmax, gsel)
    g2sq = g2 * g2
    ones_k = jnp.ones((K2, 1), jnp.float32)
    cnt_tot = cnt.sum(axis=1, keepdims=True)
    q = jax.lax.dot_general(cnt, p, _DN,
                            preferred_element_type=jnp.float32)
    s1 = _KNN * p.sum(axis=1, keepdims=True) + jax.lax.dot_general(
        g2, cnt_tot, _D0, preferred_element_type=jnp.float32)
    s2 = (_KNN * (p * p).sum(axis=1, keepdims=True)
          + jax.lax.dot_general(g2sq, cnt_tot, _D0,
                                preferred_element_type=jnp.float32)
          + 2.0 * jax.lax.dot_general(g2 * q, ones_k, _D0,
                                      preferred_element_type=jnp.float32))
    tot = float(BN * _KNN)
    mean = s1 / tot
    var = s2 / tot - mean * mean
    a1 = g1_ref[...] * jax.lax.rsqrt(var + 1e-5)
    b1 = b1_ref[...] - a1 * mean
    y = jnp.maximum(a1 * (gmax + p) + b1, 0.0)
    for b in range(_B):
        out_ref[b] = x_ref[b] + y[:, b * N:(b + 1) * N]


@jax.jit
def kernel(x, edge, W0, gamma0, beta0, anchor, sigma_p, W1, gamma1, beta1):
    B, C, H, W = x.shape
    N = H * W
    x3 = x.reshape(B, C, N)
    eg3 = edge.reshape(B, 1, N)
    fi_stack, nodes_feat = pl.pallas_call(
        _stage_a,
        out_shape=(jax.ShapeDtypeStruct((B, _KNN, N), jnp.int32),
                   jax.ShapeDtypeStruct((B * _NODE, C), jnp.float32)),
    )(x3, eg3, W0, gamma0[:, None], beta0[:, None], anchor, sigma_p)
    fi5 = jnp.concatenate(
        [fi_stack[b].reshape(_KNN * N).reshape(N, _KNN).T for b in range(B)],
        axis=1)
    out3 = pl.pallas_call(
        _stage_b2,
        out_shape=jax.ShapeDtypeStruct((B, C, N), jnp.float32),
    )(x3, nodes_feat, W1, fi5, gamma1[:, None], beta1[:, None])
    return out3.reshape(B, C, H, W)
